# baseline (speedup 1.0000x reference)
import functools
import math

import jax
import jax.numpy as jnp
from jax import lax
from jax.experimental import pallas as pl
from jax.experimental.pallas import tpu as pltpu
from jax.experimental.pallas import tpu_sc as plsc

_I32_MIN = -(2**31)
_NROWS = 512
_V = 8192
_K = math.ceil((1.0 - 0.9) * _V)
_NV = _V // 16
_NBINS = 64
_NGRP = _NBINS // 16
_UNROLL = 8
_NSEG = 16
_SEGV = _NV // _NSEG
_SEGCAP = _SEGV * 16 + 16
_FAST_CAP = 240


def _digit(v):
    t = lax.convert_element_type(v * 8.0, jnp.int32)
    return jnp.clip(t + 32, 0, _NBINS - 1)


def _key_of(v):
    b = lax.bitcast_convert_type(v, jnp.int32)
    return jnp.where(b < 0, _I32_MIN - b, b)


def _sc_body(x_hbm, o_hbm, xv0, xv1, ov0, ov1, cseg, dbuf, hist,
             si0, si1, so0, so1):
    wid = lax.axis_index("s") * 2 + lax.axis_index("c")
    ii = lax.broadcasted_iota(jnp.int32, (16,), 0)
    ones16 = jnp.full((16,), 1, jnp.int32)
    ninf16 = jnp.full((16,), -jnp.inf, jnp.float32)

    def row_slice(rr):
        return x_hbm.at[pl.ds((wid * 16 + rr) * _V, _V)]

    def out_slice(rr):
        return o_hbm.at[pl.ds((wid * 16 + rr) * _V, _V)]

    def select_row(xv, ov):
        for c in range(16):
            dbuf[pl.ds(c * 16, 16)] = ninf16

        @plsc.parallel_loop(0, _NBINS * 16 // 16, unroll=_UNROLL)
        def zero_hist(i):
            hist[pl.ds(i * 16, 16)] = jnp.zeros((16,), jnp.int32)

        @plsc.parallel_loop(0, _NV, unroll=_UNROLL)
        def pass_a(j):
            v = xv[pl.ds(j * 16, 16)]
            addr = lax.shift_left(_digit(v), 4) + ii
            plsc.addupdate_scatter(hist, [addr], ones16)

        cum = jnp.int32(0)
        found = jnp.int32(0)
        b0 = jnp.int32(0)
        r1 = jnp.int32(1)
        for g in range(_NGRP - 1, -1, -1):
            mg = jnp.zeros((16,), jnp.int32)
            for b in range(16):
                s_b = jnp.sum(hist[pl.ds((16 * g + b) * 16, 16)])
                mg = jnp.where(ii == b, s_b, mg)
            rev = lax.rev(mg, (0,))
            cs = plsc.cumsum(rev)
            tot = jnp.max(cs)
            hit = cs >= (_K - cum)
            p = jnp.max(plsc.all_reduce_ffs(hit))
            in_this = jnp.logical_and(found == 0, cum + tot >= _K)
            cnt_d = jnp.sum(jnp.where(ii == p, rev, 0))
            cs_p = jnp.sum(jnp.where(ii == p, cs, 0))
            cum_above = cum + cs_p - cnt_d
            b0 = jnp.where(in_this, 16 * g + 15 - p, b0)
            r1 = jnp.where(in_this, _K - cum_above, r1)
            found = jnp.where(in_this, jnp.int32(1), found)
            cum = cum + tot

        def pass_c(j, ptrs):
            out = []
            for s in range(_NSEG):
                v = xv[pl.ds((s * _SEGV + j) * 16, 16)]
                m = _digit(v) == b0
                plsc.store_compressed(
                    cseg.at[pl.ds(s * _SEGCAP + ptrs[s], 16)], v, mask=m)
                out.append(ptrs[s] +
                           jnp.max(plsc.all_reduce_population_count(m)))
            return tuple(out)

        ptrs = plsc.parallel_loop(0, _SEGV,
                                  carry=(jnp.int32(0),) * _NSEG)(pass_c)

        n1 = jnp.int32(0)
        for s in range(_NSEG):
            ns = ptrs[s]
            base = n1

            def cp(t, _):
                dbuf[pl.ds(base + t * 16, 16)] = (
                    cseg[pl.ds(s * _SEGCAP + t * 16, 16)])
                return 0

            lax.fori_loop(0, lax.shift_right_logical(ns + 15, 4), cp, 0)
            n1 = n1 + ns
        for c in range(4):
            dbuf[pl.ds(n1 + 16 * c, 16)] = ninf16

        def search_fast():
            kvs = [_key_of(dbuf[pl.ds(c * 16, 16)])
                   for c in range(16)]
            t_u = jnp.int32(0)
            for bit in range(31, -1, -1):
                bconst = -(1 << 31) if bit == 31 else (1 << bit)
                cand_u = t_u | jnp.int32(bconst)
                cand_s = cand_u ^ _I32_MIN
                acc = jnp.zeros((16,), jnp.int32)
                for c in range(16):
                    acc = acc + (kvs[c] >= cand_s).astype(jnp.int32)
                t_u = jnp.where(jnp.sum(acc) >= r1, cand_u, t_u)
            t_s = t_u ^ _I32_MIN
            a_gt = jnp.zeros((16,), jnp.int32)
            a_eq = jnp.zeros((16,), jnp.int32)
            for c in range(16):
                a_gt = a_gt + (kvs[c] > t_s).astype(jnp.int32)
                a_eq = a_eq + (kvs[c] == t_s).astype(jnp.int32)
            return t_s, jnp.sum(a_gt), jnp.sum(a_eq)

        def search_slow():
            nv4 = lax.shift_right_logical(n1 + 63, 6)

            def bs(i, t_u):
                cand_u = t_u | lax.shift_left(jnp.int32(1), 31 - i)
                cand_s = cand_u ^ _I32_MIN

                def cnt_body(j, acc):
                    for c in range(4):
                        kv = _key_of(dbuf[pl.ds(j * 64 + c * 16, 16)])
                        acc = acc + (kv >= cand_s).astype(jnp.int32)
                    return acc

                acc = lax.fori_loop(0, nv4, cnt_body,
                                    jnp.zeros((16,), jnp.int32))
                return jnp.where(jnp.sum(acc) >= r1, cand_u, t_u)

            t_u = lax.fori_loop(0, 32, bs, jnp.int32(0))
            t_s = t_u ^ _I32_MIN

            def stats_body(j, accs):
                a_gt, a_eq = accs
                for c in range(4):
                    kv = _key_of(dbuf[pl.ds(j * 64 + c * 16, 16)])
                    a_gt = a_gt + (kv > t_s).astype(jnp.int32)
                    a_eq = a_eq + (kv == t_s).astype(jnp.int32)
                return a_gt, a_eq

            z16 = jnp.zeros((16,), jnp.int32)
            a_gt, a_eq = lax.fori_loop(0, nv4, stats_body, (z16, z16))
            return t_s, jnp.sum(a_gt), jnp.sum(a_eq)

        t_s, c_gt, n_eq = lax.cond(n1 <= _FAST_CAP, search_fast, search_slow)
        e = r1 - c_gt

        @pl.when(e == n_eq)
        def _fast():
            @plsc.parallel_loop(0, _NV, unroll=_UNROLL)
            def out_b(j):
                sl = pl.ds(j * 16, 16)
                v = xv[sl]
                ov[sl] = jnp.where(_key_of(v) >= t_s, v, -jnp.inf)

        @pl.when(e != n_eq)
        def _slow():
            def out_b(j, run):
                sl = pl.ds(j * 16, 16)
                v = xv[sl]
                key = _key_of(v)
                eqm = key == t_s
                cs = plsc.cumsum(eqm.astype(jnp.int32))
                keep = (key > t_s) | (eqm & ((run + cs) <= e))
                ov[sl] = jnp.where(keep, v, -jnp.inf)
                return run + plsc.all_reduce_population_count(eqm)

            plsc.parallel_loop(0, _NV, unroll=_UNROLL,
                               carry=jnp.zeros((16,), jnp.int32))(out_b)

    pltpu.async_copy(row_slice(0), xv0, si0)

    def chunk(i, _):
        r0 = 2 * i
        pltpu.async_copy(row_slice(r0 + 1), xv1, si1)
        pltpu.make_async_copy(row_slice(r0), xv0, si0).wait()

        @pl.when(i > 0)
        def _w0():
            pltpu.make_async_copy(ov0, out_slice(2 * i - 2), so0).wait()

        select_row(xv0, ov0)
        pltpu.async_copy(ov0, out_slice(r0), so0)

        @pl.when(i < 7)
        def _n1():
            pltpu.async_copy(row_slice(r0 + 2), xv0, si0)

        pltpu.make_async_copy(row_slice(r0 + 1), xv1, si1).wait()

        @pl.when(i > 0)
        def _w1():
            pltpu.make_async_copy(ov1, out_slice(2 * i - 1), so1).wait()

        select_row(xv1, ov1)
        pltpu.async_copy(ov1, out_slice(r0 + 1), so1)
        return 0

    lax.fori_loop(0, 8, chunk, 0)
    pltpu.make_async_copy(ov0, out_slice(14), so0).wait()
    pltpu.make_async_copy(ov1, out_slice(15), so1).wait()


@jax.jit
def _sc_topk_mask(flat):
    mesh = plsc.VectorSubcoreMesh(core_axis_name="c", subcore_axis_name="s")
    return pl.kernel(
        _sc_body,
        out_type=jax.ShapeDtypeStruct((_NROWS * _V,), jnp.float32),
        mesh=mesh,
        compiler_params=pltpu.CompilerParams(needs_layout_passes=False),
        scratch_types=[
            pltpu.VMEM((_V,), jnp.float32),
            pltpu.VMEM((_V,), jnp.float32),
            pltpu.VMEM((_V,), jnp.float32),
            pltpu.VMEM((_V,), jnp.float32),
            pltpu.VMEM((_NSEG * _SEGCAP,), jnp.float32),
            pltpu.VMEM((_V + 64,), jnp.float32),
            pltpu.VMEM((_NBINS * 16,), jnp.int32),
            pltpu.SemaphoreType.DMA,
            pltpu.SemaphoreType.DMA,
            pltpu.SemaphoreType.DMA,
            pltpu.SemaphoreType.DMA,
        ],
    )(flat)


def kernel(logits):
    B, S, V = logits.shape
    out = _sc_topk_mask(logits.reshape(-1))
    return out.reshape(B, S, V)


# speedup vs baseline: 1.2462x; 1.2462x over previous
import functools
import math

import jax
import jax.numpy as jnp
from jax import lax
from jax.experimental import pallas as pl
from jax.experimental.pallas import tpu as pltpu
from jax.experimental.pallas import tpu_sc as plsc

_I32_MIN = -(2**31)
_NROWS = 512
_V = 8192
_K = math.ceil((1.0 - 0.9) * _V)
_NV = _V // 16
_NBINS = 64
_NGRP = _NBINS // 16
_UNROLL = 8
_NSEG = 4
_SEGV = _NV // _NSEG
_SEGCAP = _SEGV * 16 + 16
_FAST_CAP = 240


def _digit(v):
    t = lax.convert_element_type(v * 8.0, jnp.int32)
    return jnp.clip(t + 32, 0, _NBINS - 1)


def _key_of(v):
    b = lax.bitcast_convert_type(v, jnp.int32)
    return jnp.where(b < 0, _I32_MIN - b, b)


def _sc_body(x_hbm, o_hbm, xv0, xv1, ov0, ov1, cseg, dbuf, hist,
             si0, si1, so0, so1):
    wid = lax.axis_index("s") * 2 + lax.axis_index("c")
    ii = lax.broadcasted_iota(jnp.int32, (16,), 0)
    ones16 = jnp.full((16,), 1, jnp.int32)
    ninf16 = jnp.full((16,), -jnp.inf, jnp.float32)

    def row_slice(rr):
        return x_hbm.at[pl.ds((wid * 16 + rr) * _V, _V)]

    def out_slice(rr):
        return o_hbm.at[pl.ds((wid * 16 + rr) * _V, _V)]

    def select_row(xv, ov):
        for c in range(16):
            dbuf[pl.ds(c * 16, 16)] = ninf16

        @plsc.parallel_loop(0, _NBINS * 16 // 16, unroll=_UNROLL)
        def zero_hist(i):
            hist[pl.ds(i * 16, 16)] = jnp.zeros((16,), jnp.int32)

        @plsc.parallel_loop(0, _NV, unroll=_UNROLL)
        def pass_a(j):
            v = xv[pl.ds(j * 16, 16)]
            addr = lax.shift_left(_digit(v), 4) + ii
            plsc.addupdate_scatter(hist, [addr], ones16)

        cum = jnp.int32(0)
        found = jnp.int32(0)
        b0 = jnp.int32(0)
        r1 = jnp.int32(1)
        for g in range(_NGRP - 1, -1, -1):
            mg = jnp.zeros((16,), jnp.int32)
            for b in range(16):
                s_b = jnp.sum(hist[pl.ds((16 * g + b) * 16, 16)])
                mg = jnp.where(ii == b, s_b, mg)
            rev = lax.rev(mg, (0,))
            cs = plsc.cumsum(rev)
            tot = jnp.max(cs)
            hit = cs >= (_K - cum)
            p = jnp.max(plsc.all_reduce_ffs(hit))
            in_this = jnp.logical_and(found == 0, cum + tot >= _K)
            cnt_d = jnp.sum(jnp.where(ii == p, rev, 0))
            cs_p = jnp.sum(jnp.where(ii == p, cs, 0))
            cum_above = cum + cs_p - cnt_d
            b0 = jnp.where(in_this, 16 * g + 15 - p, b0)
            r1 = jnp.where(in_this, _K - cum_above, r1)
            found = jnp.where(in_this, jnp.int32(1), found)
            cum = cum + tot

        def pass_c(j, ptrs):
            out = []
            for s in range(_NSEG):
                v = xv[pl.ds((s * _SEGV + j) * 16, 16)]
                m = _digit(v) == b0
                plsc.store_compressed(
                    cseg.at[pl.ds(s * _SEGCAP + ptrs[s], 16)], v, mask=m)
                out.append(ptrs[s] +
                           jnp.max(plsc.all_reduce_population_count(m)))
            return tuple(out)

        ptrs = plsc.parallel_loop(0, _SEGV,
                                  carry=(jnp.int32(0),) * _NSEG)(pass_c)

        n1 = jnp.int32(0)
        for s in range(_NSEG):
            ns = ptrs[s]
            base = n1

            def cp(t, _):
                dbuf[pl.ds(base + t * 16, 16)] = (
                    cseg[pl.ds(s * _SEGCAP + t * 16, 16)])
                return 0

            lax.fori_loop(0, lax.shift_right_logical(ns + 15, 4), cp, 0)
            n1 = n1 + ns
        for c in range(4):
            dbuf[pl.ds(n1 + 16 * c, 16)] = ninf16

        def search_fast():
            kvs = [_key_of(dbuf[pl.ds(c * 16, 16)])
                   for c in range(16)]
            t_u = jnp.int32(0)
            for bit in range(31, -1, -1):
                bconst = -(1 << 31) if bit == 31 else (1 << bit)
                cand_u = t_u | jnp.int32(bconst)
                cand_s = cand_u ^ _I32_MIN
                acc = jnp.zeros((16,), jnp.int32)
                for c in range(16):
                    acc = acc + (kvs[c] >= cand_s).astype(jnp.int32)
                t_u = jnp.where(jnp.sum(acc) >= r1, cand_u, t_u)
            t_s = t_u ^ _I32_MIN
            a_gt = jnp.zeros((16,), jnp.int32)
            a_eq = jnp.zeros((16,), jnp.int32)
            for c in range(16):
                a_gt = a_gt + (kvs[c] > t_s).astype(jnp.int32)
                a_eq = a_eq + (kvs[c] == t_s).astype(jnp.int32)
            return t_s, jnp.sum(a_gt), jnp.sum(a_eq)

        def search_slow():
            nv4 = lax.shift_right_logical(n1 + 63, 6)

            def bs(i, t_u):
                cand_u = t_u | lax.shift_left(jnp.int32(1), 31 - i)
                cand_s = cand_u ^ _I32_MIN

                def cnt_body(j, acc):
                    for c in range(4):
                        kv = _key_of(dbuf[pl.ds(j * 64 + c * 16, 16)])
                        acc = acc + (kv >= cand_s).astype(jnp.int32)
                    return acc

                acc = lax.fori_loop(0, nv4, cnt_body,
                                    jnp.zeros((16,), jnp.int32))
                return jnp.where(jnp.sum(acc) >= r1, cand_u, t_u)

            t_u = lax.fori_loop(0, 32, bs, jnp.int32(0))
            t_s = t_u ^ _I32_MIN

            def stats_body(j, accs):
                a_gt, a_eq = accs
                for c in range(4):
                    kv = _key_of(dbuf[pl.ds(j * 64 + c * 16, 16)])
                    a_gt = a_gt + (kv > t_s).astype(jnp.int32)
                    a_eq = a_eq + (kv == t_s).astype(jnp.int32)
                return a_gt, a_eq

            z16 = jnp.zeros((16,), jnp.int32)
            a_gt, a_eq = lax.fori_loop(0, nv4, stats_body, (z16, z16))
            return t_s, jnp.sum(a_gt), jnp.sum(a_eq)

        t_s, c_gt, n_eq = lax.cond(n1 <= _FAST_CAP, search_fast, search_slow)
        e = r1 - c_gt

        @pl.when(e == n_eq)
        def _fast():
            @plsc.parallel_loop(0, _NV, unroll=_UNROLL)
            def out_b(j):
                sl = pl.ds(j * 16, 16)
                v = xv[sl]
                ov[sl] = jnp.where(_key_of(v) >= t_s, v, -jnp.inf)

        @pl.when(e != n_eq)
        def _slow():
            def out_b(j, run):
                sl = pl.ds(j * 16, 16)
                v = xv[sl]
                key = _key_of(v)
                eqm = key == t_s
                cs = plsc.cumsum(eqm.astype(jnp.int32))
                keep = (key > t_s) | (eqm & ((run + cs) <= e))
                ov[sl] = jnp.where(keep, v, -jnp.inf)
                return run + plsc.all_reduce_population_count(eqm)

            plsc.parallel_loop(0, _NV, unroll=_UNROLL,
                               carry=jnp.zeros((16,), jnp.int32))(out_b)

    pltpu.async_copy(row_slice(0), xv0, si0)

    def chunk(i, _):
        r0 = 2 * i
        pltpu.async_copy(row_slice(r0 + 1), xv1, si1)
        pltpu.make_async_copy(row_slice(r0), xv0, si0).wait()

        @pl.when(i > 0)
        def _w0():
            pltpu.make_async_copy(ov0, out_slice(2 * i - 2), so0).wait()

        select_row(xv0, ov0)
        pltpu.async_copy(ov0, out_slice(r0), so0)

        @pl.when(i < 7)
        def _n1():
            pltpu.async_copy(row_slice(r0 + 2), xv0, si0)

        pltpu.make_async_copy(row_slice(r0 + 1), xv1, si1).wait()

        @pl.when(i > 0)
        def _w1():
            pltpu.make_async_copy(ov1, out_slice(2 * i - 1), so1).wait()

        select_row(xv1, ov1)
        pltpu.async_copy(ov1, out_slice(r0 + 1), so1)
        return 0

    lax.fori_loop(0, 8, chunk, 0)
    pltpu.make_async_copy(ov0, out_slice(14), so0).wait()
    pltpu.make_async_copy(ov1, out_slice(15), so1).wait()


@jax.jit
def _sc_topk_mask(flat):
    mesh = plsc.VectorSubcoreMesh(core_axis_name="c", subcore_axis_name="s")
    return pl.kernel(
        _sc_body,
        out_type=jax.ShapeDtypeStruct((_NROWS * _V,), jnp.float32),
        mesh=mesh,
        compiler_params=pltpu.CompilerParams(needs_layout_passes=False),
        scratch_types=[
            pltpu.VMEM((_V,), jnp.float32),
            pltpu.VMEM((_V,), jnp.float32),
            pltpu.VMEM((_V,), jnp.float32),
            pltpu.VMEM((_V,), jnp.float32),
            pltpu.VMEM((_NSEG * _SEGCAP,), jnp.float32),
            pltpu.VMEM((_V + 64,), jnp.float32),
            pltpu.VMEM((_NBINS * 16,), jnp.int32),
            pltpu.SemaphoreType.DMA,
            pltpu.SemaphoreType.DMA,
            pltpu.SemaphoreType.DMA,
            pltpu.SemaphoreType.DMA,
        ],
    )(flat)


def kernel(logits):
    B, S, V = logits.shape
    out = _sc_topk_mask(logits.reshape(-1))
    return out.reshape(B, S, V)


# speedup vs baseline: 1.2993x; 1.0426x over previous
import functools
import math

import jax
import jax.numpy as jnp
from jax import lax
from jax.experimental import pallas as pl
from jax.experimental.pallas import tpu as pltpu
from jax.experimental.pallas import tpu_sc as plsc

_I32_MIN = -(2**31)
_NROWS = 512
_V = 8192
_K = math.ceil((1.0 - 0.9) * _V)
_NV = _V // 16
_NBINS = 64
_NGRP = _NBINS // 16
_UNROLL = 8
_FAST_CAP = 240


def _digit(v):
    t = lax.convert_element_type(v * 8.0, jnp.int32)
    return jnp.clip(t + 32, 0, _NBINS - 1)


def _key_of(v):
    b = lax.bitcast_convert_type(v, jnp.int32)
    return jnp.where(b < 0, _I32_MIN - b, b)


def _sc_body(x_hbm, o_hbm, xv0, xv1, ov0, ov1, cbuf, hist,
             si0, si1, so0, so1):
    wid = lax.axis_index("s") * 2 + lax.axis_index("c")
    ii = lax.broadcasted_iota(jnp.int32, (16,), 0)
    ones16 = jnp.full((16,), 1, jnp.int32)
    pad16 = jnp.full((16,), _I32_MIN, jnp.int32)

    def row_slice(rr):
        return x_hbm.at[pl.ds((wid * 16 + rr) * _V, _V)]

    def out_slice(rr):
        return o_hbm.at[pl.ds((wid * 16 + rr) * _V, _V)]

    def select_row(xv, ov):
        for c in range(16):
            cbuf[pl.ds(c * 16, 16)] = pad16

        @plsc.parallel_loop(0, _NBINS * 16 // 16, unroll=_UNROLL)
        def zero_hist(i):
            hist[pl.ds(i * 16, 16)] = jnp.zeros((16,), jnp.int32)

        @plsc.parallel_loop(0, _NV, unroll=_UNROLL)
        def pass_a(j):
            v = xv[pl.ds(j * 16, 16)]
            addr = lax.shift_left(_digit(v), 4) + ii
            plsc.addupdate_scatter(hist, [addr], ones16)

        cum = jnp.int32(0)
        found = jnp.int32(0)
        b0 = jnp.int32(0)
        r1 = jnp.int32(1)
        for g in range(_NGRP - 1, -1, -1):
            mg = jnp.zeros((16,), jnp.int32)
            for b in range(16):
                s_b = jnp.sum(hist[pl.ds((16 * g + b) * 16, 16)])
                mg = jnp.where(ii == b, s_b, mg)
            rev = lax.rev(mg, (0,))
            cs = plsc.cumsum(rev)
            tot = jnp.max(cs)
            hit = cs >= (_K - cum)
            p = jnp.max(plsc.all_reduce_ffs(hit))
            in_this = jnp.logical_and(found == 0, cum + tot >= _K)
            cnt_d = jnp.sum(jnp.where(ii == p, rev, 0))
            cs_p = jnp.sum(jnp.where(ii == p, cs, 0))
            cum_above = cum + cs_p - cnt_d
            b0 = jnp.where(in_this, 16 * g + 15 - p, b0)
            r1 = jnp.where(in_this, _K - cum_above, r1)
            found = jnp.where(in_this, jnp.int32(1), found)
            cum = cum + tot

        def pass_c(j, ptr):
            v = xv[pl.ds(j * 16, 16)]
            m = _digit(v) == b0
            plsc.store_compressed(cbuf.at[pl.ds(ptr, 16)], _key_of(v),
                                  mask=m)
            return ptr + jnp.max(plsc.all_reduce_population_count(m))

        n1 = plsc.parallel_loop(0, _NV, unroll=_UNROLL,
                                carry=jnp.int32(0))(pass_c)

        def search_fast():
            kvs = [cbuf[pl.ds(c * 16, 16)] for c in range(16)]
            t_u = jnp.int32(0)
            for bit in range(31, -1, -1):
                bconst = -(1 << 31) if bit == 31 else (1 << bit)
                cand_u = t_u | jnp.int32(bconst)
                cand_s = cand_u ^ _I32_MIN
                acc = jnp.zeros((16,), jnp.int32)
                for c in range(16):
                    acc = acc + (kvs[c] >= cand_s).astype(jnp.int32)
                t_u = jnp.where(jnp.sum(acc) >= r1, cand_u, t_u)
            t_s = t_u ^ _I32_MIN
            a_gt = jnp.zeros((16,), jnp.int32)
            a_eq = jnp.zeros((16,), jnp.int32)
            for c in range(16):
                a_gt = a_gt + (kvs[c] > t_s).astype(jnp.int32)
                a_eq = a_eq + (kvs[c] == t_s).astype(jnp.int32)
            return t_s, jnp.sum(a_gt), jnp.sum(a_eq)

        def search_slow():
            for c in range(4):
                cbuf[pl.ds(n1 + 16 * c, 16)] = pad16
            nv4 = lax.shift_right_logical(n1 + 63, 6)

            def bs(i, t_u):
                cand_u = t_u | lax.shift_left(jnp.int32(1), 31 - i)
                cand_s = cand_u ^ _I32_MIN

                def cnt_body(j, acc):
                    for c in range(4):
                        kv = cbuf[pl.ds(j * 64 + c * 16, 16)]
                        acc = acc + (kv >= cand_s).astype(jnp.int32)
                    return acc

                acc = lax.fori_loop(0, nv4, cnt_body,
                                    jnp.zeros((16,), jnp.int32))
                return jnp.where(jnp.sum(acc) >= r1, cand_u, t_u)

            t_u = lax.fori_loop(0, 32, bs, jnp.int32(0))
            t_s = t_u ^ _I32_MIN

            def stats_body(j, accs):
                a_gt, a_eq = accs
                for c in range(4):
                    kv = cbuf[pl.ds(j * 64 + c * 16, 16)]
                    a_gt = a_gt + (kv > t_s).astype(jnp.int32)
                    a_eq = a_eq + (kv == t_s).astype(jnp.int32)
                return a_gt, a_eq

            z16 = jnp.zeros((16,), jnp.int32)
            a_gt, a_eq = lax.fori_loop(0, nv4, stats_body, (z16, z16))
            return t_s, jnp.sum(a_gt), jnp.sum(a_eq)

        t_s, c_gt, n_eq = lax.cond(n1 <= _FAST_CAP, search_fast, search_slow)
        e = r1 - c_gt

        @pl.when(e == n_eq)
        def _fast():
            @plsc.parallel_loop(0, _NV, unroll=_UNROLL)
            def out_b(j):
                sl = pl.ds(j * 16, 16)
                v = xv[sl]
                ov[sl] = jnp.where(_key_of(v) >= t_s, v, -jnp.inf)

        @pl.when(e != n_eq)
        def _slow():
            def out_b(j, run):
                sl = pl.ds(j * 16, 16)
                v = xv[sl]
                key = _key_of(v)
                eqm = key == t_s
                cs = plsc.cumsum(eqm.astype(jnp.int32))
                keep = (key > t_s) | (eqm & ((run + cs) <= e))
                ov[sl] = jnp.where(keep, v, -jnp.inf)
                return run + plsc.all_reduce_population_count(eqm)

            plsc.parallel_loop(0, _NV, unroll=_UNROLL,
                               carry=jnp.zeros((16,), jnp.int32))(out_b)

    pltpu.async_copy(row_slice(0), xv0, si0)

    def chunk(i, _):
        r0 = 2 * i
        pltpu.async_copy(row_slice(r0 + 1), xv1, si1)
        pltpu.make_async_copy(row_slice(r0), xv0, si0).wait()

        @pl.when(i > 0)
        def _w0():
            pltpu.make_async_copy(ov0, out_slice(2 * i - 2), so0).wait()

        select_row(xv0, ov0)
        pltpu.async_copy(ov0, out_slice(r0), so0)

        @pl.when(i < 7)
        def _n1():
            pltpu.async_copy(row_slice(r0 + 2), xv0, si0)

        pltpu.make_async_copy(row_slice(r0 + 1), xv1, si1).wait()

        @pl.when(i > 0)
        def _w1():
            pltpu.make_async_copy(ov1, out_slice(2 * i - 1), so1).wait()

        select_row(xv1, ov1)
        pltpu.async_copy(ov1, out_slice(r0 + 1), so1)
        return 0

    lax.fori_loop(0, 8, chunk, 0)
    pltpu.make_async_copy(ov0, out_slice(14), so0).wait()
    pltpu.make_async_copy(ov1, out_slice(15), so1).wait()


@jax.jit
def _sc_topk_mask(flat):
    mesh = plsc.VectorSubcoreMesh(core_axis_name="c", subcore_axis_name="s")
    return pl.kernel(
        _sc_body,
        out_type=jax.ShapeDtypeStruct((_NROWS * _V,), jnp.float32),
        mesh=mesh,
        compiler_params=pltpu.CompilerParams(needs_layout_passes=False),
        scratch_types=[
            pltpu.VMEM((_V,), jnp.float32),
            pltpu.VMEM((_V,), jnp.float32),
            pltpu.VMEM((_V,), jnp.float32),
            pltpu.VMEM((_V,), jnp.float32),
            pltpu.VMEM((_V + 64,), jnp.int32),
            pltpu.VMEM((_NBINS * 16,), jnp.int32),
            pltpu.SemaphoreType.DMA,
            pltpu.SemaphoreType.DMA,
            pltpu.SemaphoreType.DMA,
            pltpu.SemaphoreType.DMA,
        ],
    )(flat)


def kernel(logits):
    B, S, V = logits.shape
    out = _sc_topk_mask(logits.reshape(-1))
    return out.reshape(B, S, V)
